# baseline (speedup 1.0000x reference)
import functools

import jax
import jax.numpy as jnp
from jax import lax
from jax.experimental import pallas as pl
from jax.experimental.pallas import tpu as pltpu
from jax.experimental.pallas import tpu_sc as plsc

_L = 16
_H = 1024
_W = 1024
_C = 32
_CP = 33
_NW = 32
_N = _H * _W
_CH = 128

_UV_PREFETCH = False
_FIRE_GATHERS = True
_DO_BLEND = True

_mesh = plsc.VectorSubcoreMesh(core_axis_name="c", subcore_axis_name="s")
_params = pltpu.CompilerParams(
    use_tc_tiling_on_sc=False, needs_layout_passes=False
)

_BCAST_DNUMS = lax.GatherDimensionNumbers(
    offset_dims=(), collapsed_slice_dims=(0,), start_index_map=(0,)
)


def _lane_bcast(vec, lane):
    sel = jnp.full((_L,), lane, jnp.int32)
    return lax.gather(
        vec,
        sel[:, None],
        _BCAST_DNUMS,
        slice_sizes=(1,),
        mode=lax.GatherScatterMode.PROMISE_IN_BOUNDS,
    )


@functools.partial(
    pl.kernel,
    out_type=jax.ShapeDtypeStruct((_N, 2 * _C), jnp.float32),
    mesh=_mesh,
    scratch_types=[
        [pltpu.VMEM((4, 2, 8, 128), jnp.float32) for _ in range(2)],
        pltpu.VMEM((_W + 1, _CP), jnp.float32),
        [pltpu.VMEM((_W // 2, 2 * _C), jnp.float32) for _ in range(2)],
        [pltpu.SemaphoreType.DMA for _ in range(2)],
        [pltpu.SemaphoreType.DMA for _ in range(2)],
    ],
    compiler_params=_params,
)
def _convert(tex5_hbm, table_hbm, in_v, t_v, c_v, isems, osems):
    cid = lax.axis_index("c")
    sid = lax.axis_index("s")
    wid = sid * 2 + cid
    planes = _H // _NW
    y0 = wid * planes

    def fire_in(y, q, b):
        for ct in range(4):
            pltpu.async_copy(tex5_hbm.at[y, ct, pl.ds(q * 2, 2)],
                             in_v[b].at[ct], isems[b])

    def wait_in(b):
        for ct in range(4):
            pltpu.make_async_copy(tex5_hbm.at[0, 0, pl.ds(0, 2)],
                                  in_v[b].at[ct], isems[b]).wait()

    fire_in(y0, 0, 0)

    def plane_body(i, carry):
        y = y0 + i
        for q in range(4):
            b = q % 2
            wait_in(b)

            if q < 3:
                fire_in(y, q + 1, 1 - b)
            else:
                @pl.when(i + 1 < planes)
                def _(y=y, b=b):
                    fire_in(y + 1, 0, 1 - b)

            @plsc.parallel_loop(0, 256 // _L)
            def _grp(g):
                xt = g // 8
                xg = g - xt * 8
                pvec = (q * 256) + g * _L + lax.iota(jnp.int32, _L)
                for ch in range(_C):
                    val = in_v[b][ch // 8, xt, ch % 8, pl.ds(xg * _L, _L)]
                    cvec = jnp.full((_L,), ch, jnp.int32)
                    plsc.store_scatter(t_v, [pvec, cvec], val)

        for half in range(_C // _L):
            t_v[_W, pl.ds(half * _L, _L)] = t_v[0, pl.ds(half * _L, _L)]

        for hh in range(2):
            @pl.when(i >= 1)
            def _():
                pltpu.make_async_copy(c_v[hh],
                                      table_hbm.at[pl.ds(0, _W // 2)],
                                      osems[hh]).wait()

            @plsc.parallel_loop(0, _W // 2)
            def _cmp(pl_):
                p = hh * (_W // 2) + pl_
                for half in range(_C // _L):
                    s = half * _L
                    c_v[hh][pl_, pl.ds(s, _L)] = t_v[p, pl.ds(s, _L)]
                    c_v[hh][pl_, pl.ds(_C + s, _L)] = t_v[p + 1, pl.ds(s, _L)]

            pltpu.async_copy(
                c_v[hh],
                table_hbm.at[pl.ds(y * _W + hh * (_W // 2), _W // 2)],
                osems[hh])
        return carry

    lax.fori_loop(0, planes, plane_body, 0)
    for hh in range(2):
        pltpu.make_async_copy(c_v[hh], table_hbm.at[pl.ds(0, _W // 2)],
                              osems[hh]).wait()


@functools.partial(
    pl.kernel,
    out_type=jax.ShapeDtypeStruct((_H, 4, 8, 8, 128), jnp.float32),
    mesh=_mesh,
    scratch_types=[
        [pltpu.VMEM((_CH,), jnp.float32) for _ in range(2)],
        [pltpu.VMEM((_CH,), jnp.float32) for _ in range(2)],
        [pltpu.VMEM((_CH,), jnp.float32) for _ in range(2)],
        [pltpu.VMEM((_CH,), jnp.float32) for _ in range(2)],
        [pltpu.VMEM((2, _CH), jnp.int32) for _ in range(2)],
        [pltpu.VMEM((2, _CH, 2 * _C), jnp.float32) for _ in range(2)],
        pltpu.VMEM((_CH, _CP), jnp.float32),
        [pltpu.VMEM((4, 8, 128), jnp.float32) for _ in range(2)],
        [pltpu.SemaphoreType.DMA for _ in range(2)],
        [pltpu.SemaphoreType.DMA for _ in range(2)],
        [pltpu.SemaphoreType.DMA for _ in range(2)],
    ],
    compiler_params=_params,
)
def _sample(table_hbm, u5_hbm, v5_hbm, o5_hbm,
            u_v, v_v, fx_v, fy_v, idx_v, rows_v, o_p, o_t,
            usems, gsems, osems):
    cid = lax.axis_index("c")
    sid = lax.axis_index("s")
    wid = sid * 2 + cid
    rows = _H // _NW
    n_chunks = rows * 8

    def fire_uv(k, b):
        r = wid * rows + k // 8
        xt = k - (k // 8) * 8
        yt = r // 8
        yi = r - yt * 8
        pltpu.async_copy(u5_hbm.at[yt, xt, yi], u_v[b], usems[b])
        pltpu.async_copy(v5_hbm.at[yt, xt, yi], v_v[b], usems[b])

    def stage(k, b):
        if _UV_PREFETCH:
            pltpu.make_async_copy(u5_hbm.at[0, 0, 0], u_v[b], usems[b]).wait()
            pltpu.make_async_copy(v5_hbm.at[0, 0, 0], v_v[b], usems[b]).wait()
        else:
            r = wid * rows + k // 8
            xt = k - (k // 8) * 8
            yt = r // 8
            yi = r - yt * 8
            pltpu.sync_copy(u5_hbm.at[yt, xt, yi], u_v[b])
            pltpu.sync_copy(v5_hbm.at[yt, xt, yi], v_v[b])

        @plsc.parallel_loop(0, _CH // _L)
        def _idx(g):
            s = g * _L
            uu = u_v[b][pl.ds(s, _L)]
            vv = v_v[b][pl.ds(s, _L)]
            x = uu * float(_W) - 0.5
            y = vv * float(_H) - 0.5
            xi = x.astype(jnp.int32)
            yi2 = y.astype(jnp.int32)
            x0 = jnp.where(xi.astype(jnp.float32) > x, xi - 1, xi)
            y0 = jnp.where(yi2.astype(jnp.float32) > y, yi2 - 1, yi2)
            fx_v[b][pl.ds(s, _L)] = x - x0.astype(jnp.float32)
            fy_v[b][pl.ds(s, _L)] = y - y0.astype(jnp.float32)
            x0 = jnp.where(x0 < 0, x0 + _W, x0)
            y0 = jnp.where(y0 < 0, y0 + _H, y0)
            y1 = y0 + 1
            y1 = jnp.where(y1 == _H, 0, y1)
            idx_v[b][0, pl.ds(s, _L)] = y0 * _W + x0
            idx_v[b][1, pl.ds(s, _L)] = y1 * _W + x0

        if _FIRE_GATHERS:
            for c in range(2):
                pltpu.async_copy(table_hbm.at[idx_v[b].at[c]],
                                 rows_v[b].at[c], gsems[b])

    def finish(k, b, first):
        r = wid * rows + k // 8
        xt = k - (k // 8) * 8
        if _FIRE_GATHERS:
            for c in range(2):
                pltpu.make_async_copy(table_hbm.at[idx_v[b].at[c]],
                                      rows_v[b].at[c], gsems[b]).wait()

        @plsc.parallel_loop(0, (_CH // _L) if _DO_BLEND else 0)
        def _blend(g):
            s = g * _L
            fx16 = fx_v[b][pl.ds(s, _L)]
            fy16 = fy_v[b][pl.ds(s, _L)]
            for lp in range(_L):
                p = s + lp
                fxp = _lane_bcast(fx16, lp)
                fyp = _lane_bcast(fy16, lp)
                for half in range(_C // _L):
                    cs = half * _L
                    v00 = rows_v[b][0, p, pl.ds(cs, _L)]
                    v01 = rows_v[b][0, p, pl.ds(_C + cs, _L)]
                    v10 = rows_v[b][1, p, pl.ds(cs, _L)]
                    v11 = rows_v[b][1, p, pl.ds(_C + cs, _L)]
                    a = v00 + fxp * (v01 - v00)
                    c2 = v10 + fxp * (v11 - v10)
                    o_p[p, pl.ds(cs, _L)] = a + fyp * (c2 - a)

        @pl.when(jnp.logical_not(first))
        def _():
            for ct in range(4):
                pltpu.make_async_copy(o_t[b].at[ct],
                                      o5_hbm.at[0, ct, 0], osems[b]).wait()

        @plsc.parallel_loop(0, (_CH // _L) if _DO_BLEND else 0)
        def _tr(g):
            s = g * _L
            pvec = s + lax.iota(jnp.int32, _L)
            for ch in range(_C):
                cvec = jnp.full((_L,), ch, jnp.int32)
                o_t[b][ch // 8, ch % 8, pl.ds(s, _L)] = plsc.load_gather(
                    o_p, [pvec, cvec])

        for ct in range(4):
            pltpu.async_copy(o_t[b].at[ct], o5_hbm.at[r, ct, xt], osems[b])

    if _UV_PREFETCH:
        fire_uv(0, 0)
        fire_uv(1, 1)
    stage(0, 0)

    def pair_body(kk, carry):
        k = kk * 2

        if _UV_PREFETCH:
            @pl.when(k + 2 < n_chunks)
            def _():
                fire_uv(k + 2, 0)

        stage(k + 1, 1)

        if _UV_PREFETCH:
            @pl.when(k + 3 < n_chunks)
            def _():
                fire_uv(k + 3, 1)

        finish(k, 0, kk == 0)

        @pl.when(k + 2 < n_chunks)
        def _():
            stage(k + 2, 0)

        finish(k + 1, 1, kk == 0)
        return carry

    lax.fori_loop(0, n_chunks // 2, pair_body, 0)
    for b in range(2):
        for ct in range(4):
            pltpu.make_async_copy(o_t[b].at[ct], o5_hbm.at[0, ct, 0],
                                  osems[b]).wait()


def kernel(texture, u, v):
    tex5 = texture.reshape(_H, 8, 128, 4, 8).transpose(0, 3, 1, 4, 2)
    u5 = u.reshape(128, 8, 8, 128).transpose(0, 2, 1, 3)
    v5 = v.reshape(128, 8, 8, 128).transpose(0, 2, 1, 3)
    table = _convert(tex5)
    o5 = _sample(table, u5, v5)
    return o5.transpose(0, 2, 4, 1, 3).reshape(_H, _W, _C)


# speedup vs baseline: 1.2870x; 1.2870x over previous
import functools

import jax
import jax.numpy as jnp
from jax import lax
from jax.experimental import pallas as pl
from jax.experimental.pallas import tpu as pltpu
from jax.experimental.pallas import tpu_sc as plsc

_L = 16
_H = 1024
_W = 1024
_C = 32
_CP = 33
_NW = 32
_N = _H * _W
_CH = 128

_mesh = plsc.VectorSubcoreMesh(core_axis_name="c", subcore_axis_name="s")
_params = pltpu.CompilerParams(
    use_tc_tiling_on_sc=False, needs_layout_passes=False
)

_BCAST_DNUMS = lax.GatherDimensionNumbers(
    offset_dims=(), collapsed_slice_dims=(0,), start_index_map=(0,)
)


def _lane_bcast(vec, lane):
    sel = jnp.full((_L,), lane, jnp.int32)
    return lax.gather(
        vec,
        sel[:, None],
        _BCAST_DNUMS,
        slice_sizes=(1,),
        mode=lax.GatherScatterMode.PROMISE_IN_BOUNDS,
    )


@functools.partial(
    pl.kernel,
    out_type=jax.ShapeDtypeStruct((_N, 2 * _C), jnp.float32),
    mesh=_mesh,
    scratch_types=[
        [pltpu.VMEM((4, 2, 8, 128), jnp.float32) for _ in range(2)],
        pltpu.VMEM((_W + 1, _CP), jnp.float32),
        [pltpu.VMEM((_W // 2, 2 * _C), jnp.float32) for _ in range(2)],
        [pltpu.SemaphoreType.DMA for _ in range(2)],
        [pltpu.SemaphoreType.DMA for _ in range(2)],
    ],
    compiler_params=_params,
)
def _convert(tex5_hbm, table_hbm, in_v, t_v, c_v, isems, osems):
    cid = lax.axis_index("c")
    sid = lax.axis_index("s")
    wid = sid * 2 + cid
    planes = _H // _NW
    y0 = wid * planes

    def fire_in(y, q, b):
        for ct in range(4):
            pltpu.async_copy(tex5_hbm.at[y, ct, pl.ds(q * 2, 2)],
                             in_v[b].at[ct], isems[b])

    def wait_in(b):
        for ct in range(4):
            pltpu.make_async_copy(tex5_hbm.at[0, 0, pl.ds(0, 2)],
                                  in_v[b].at[ct], isems[b]).wait()

    fire_in(y0, 0, 0)

    def plane_body(i, carry):
        y = y0 + i
        for q in range(4):
            b = q % 2
            wait_in(b)

            if q < 3:
                fire_in(y, q + 1, 1 - b)
            else:
                @pl.when(i + 1 < planes)
                def _(y=y, b=b):
                    fire_in(y + 1, 0, 1 - b)

            @plsc.parallel_loop(0, 256 // _L)
            def _grp(g):
                xt = g // 8
                xg = g - xt * 8
                pvec = (q * 256) + g * _L + lax.iota(jnp.int32, _L)
                for ch in range(_C):
                    val = in_v[b][ch // 8, xt, ch % 8, pl.ds(xg * _L, _L)]
                    cvec = jnp.full((_L,), ch, jnp.int32)
                    plsc.store_scatter(t_v, [pvec, cvec], val)

        for half in range(_C // _L):
            t_v[_W, pl.ds(half * _L, _L)] = t_v[0, pl.ds(half * _L, _L)]

        for hh in range(2):
            @pl.when(i >= 1)
            def _():
                pltpu.make_async_copy(c_v[hh],
                                      table_hbm.at[pl.ds(0, _W // 2)],
                                      osems[hh]).wait()

            @plsc.parallel_loop(0, _W // 2)
            def _cmp(pl_):
                p = hh * (_W // 2) + pl_
                for half in range(_C // _L):
                    s = half * _L
                    c_v[hh][pl_, pl.ds(s, _L)] = t_v[p, pl.ds(s, _L)]
                    c_v[hh][pl_, pl.ds(_C + s, _L)] = t_v[p + 1, pl.ds(s, _L)]

            pltpu.async_copy(
                c_v[hh],
                table_hbm.at[pl.ds(y * _W + hh * (_W // 2), _W // 2)],
                osems[hh])
        return carry

    lax.fori_loop(0, planes, plane_body, 0)
    for hh in range(2):
        pltpu.make_async_copy(c_v[hh], table_hbm.at[pl.ds(0, _W // 2)],
                              osems[hh]).wait()


@functools.partial(
    pl.kernel,
    out_type=jax.ShapeDtypeStruct((_H, 4, 8, 8, 128), jnp.float32),
    mesh=_mesh,
    scratch_types=[
        pltpu.VMEM((8, _CH), jnp.float32),
        pltpu.VMEM((8, _CH), jnp.float32),
        [pltpu.VMEM((_CH,), jnp.float32) for _ in range(2)],
        [pltpu.VMEM((_CH,), jnp.float32) for _ in range(2)],
        [pltpu.VMEM((2, _CH), jnp.int32) for _ in range(2)],
        [pltpu.VMEM((2, _CH, 2 * _C), jnp.float32) for _ in range(2)],
        pltpu.VMEM((_CH, _CP), jnp.float32),
        [pltpu.VMEM((4, 8, 128), jnp.float32) for _ in range(2)],
        [pltpu.SemaphoreType.DMA for _ in range(2)],
        [pltpu.SemaphoreType.DMA for _ in range(2)],
    ],
    compiler_params=_params,
)
def _sample(table_hbm, u5_hbm, v5_hbm, o5_hbm,
            u_row, v_row, fx_v, fy_v, idx_v, rows_v, o_p, o_t,
            gsems, osems):
    cid = lax.axis_index("c")
    sid = lax.axis_index("s")
    wid = sid * 2 + cid
    rows = _H // _NW
    n_chunks = rows * 8

    def stage(k, b):
        r = wid * rows + k // 8
        xt = k - (k // 8) * 8
        yt = r // 8
        yi = r - yt * 8

        @pl.when(xt == 0)
        def _():
            pltpu.sync_copy(u5_hbm.at[yt, pl.ds(0, 8), yi], u_row)
            pltpu.sync_copy(v5_hbm.at[yt, pl.ds(0, 8), yi], v_row)

        @plsc.parallel_loop(0, _CH // _L)
        def _idx(g):
            s = g * _L
            uu = u_row[xt, pl.ds(s, _L)]
            vv = v_row[xt, pl.ds(s, _L)]
            x = uu * float(_W) - 0.5
            y = vv * float(_H) - 0.5
            xi = x.astype(jnp.int32)
            yi2 = y.astype(jnp.int32)
            x0 = jnp.where(xi.astype(jnp.float32) > x, xi - 1, xi)
            y0 = jnp.where(yi2.astype(jnp.float32) > y, yi2 - 1, yi2)
            fx_v[b][pl.ds(s, _L)] = x - x0.astype(jnp.float32)
            fy_v[b][pl.ds(s, _L)] = y - y0.astype(jnp.float32)
            x0 = jnp.where(x0 < 0, x0 + _W, x0)
            y0 = jnp.where(y0 < 0, y0 + _H, y0)
            y1 = y0 + 1
            y1 = jnp.where(y1 == _H, 0, y1)
            idx_v[b][0, pl.ds(s, _L)] = y0 * _W + x0
            idx_v[b][1, pl.ds(s, _L)] = y1 * _W + x0

        for c in range(2):
            pltpu.async_copy(table_hbm.at[idx_v[b].at[c]],
                             rows_v[b].at[c], gsems[b])

    def finish(k, b, first):
        r = wid * rows + k // 8
        xt = k - (k // 8) * 8
        for c in range(2):
            pltpu.make_async_copy(table_hbm.at[idx_v[b].at[c]],
                                  rows_v[b].at[c], gsems[b]).wait()

        @plsc.parallel_loop(0, _CH // _L)
        def _blend(g):
            s = g * _L
            fx16 = fx_v[b][pl.ds(s, _L)]
            fy16 = fy_v[b][pl.ds(s, _L)]
            for lp in range(_L):
                p = s + lp
                fxp = _lane_bcast(fx16, lp)
                fyp = _lane_bcast(fy16, lp)
                gxp = 1.0 - fxp
                gyp = 1.0 - fyp
                w00 = gxp * gyp
                w01 = fxp * gyp
                w10 = gxp * fyp
                w11 = fxp * fyp
                for half in range(_C // _L):
                    cs = half * _L
                    v00 = rows_v[b][0, p, pl.ds(cs, _L)]
                    v01 = rows_v[b][0, p, pl.ds(_C + cs, _L)]
                    v10 = rows_v[b][1, p, pl.ds(cs, _L)]
                    v11 = rows_v[b][1, p, pl.ds(_C + cs, _L)]
                    o_p[p, pl.ds(cs, _L)] = (
                        v00 * w00 + v01 * w01 + v10 * w10 + v11 * w11
                    )

        @pl.when(jnp.logical_not(first))
        def _():
            for ct in range(4):
                pltpu.make_async_copy(o_t[b].at[ct],
                                      o5_hbm.at[0, ct, 0], osems[b]).wait()

        @plsc.parallel_loop(0, _CH // _L)
        def _tr(g):
            s = g * _L
            pvec = s + lax.iota(jnp.int32, _L)
            for ch in range(_C):
                cvec = jnp.full((_L,), ch, jnp.int32)
                o_t[b][ch // 8, ch % 8, pl.ds(s, _L)] = plsc.load_gather(
                    o_p, [pvec, cvec])

        for ct in range(4):
            pltpu.async_copy(o_t[b].at[ct], o5_hbm.at[r, ct, xt], osems[b])

    stage(0, 0)

    def pair_body(kk, carry):
        k = kk * 2
        stage(k + 1, 1)
        finish(k, 0, kk == 0)

        @pl.when(k + 2 < n_chunks)
        def _():
            stage(k + 2, 0)

        finish(k + 1, 1, kk == 0)
        return carry

    lax.fori_loop(0, n_chunks // 2, pair_body, 0)
    for b in range(2):
        for ct in range(4):
            pltpu.make_async_copy(o_t[b].at[ct], o5_hbm.at[0, ct, 0],
                                  osems[b]).wait()


def kernel(texture, u, v):
    tex5 = texture.reshape(_H, 8, 128, 4, 8).transpose(0, 3, 1, 4, 2)
    u5 = u.reshape(128, 8, 8, 128).transpose(0, 2, 1, 3)
    v5 = v.reshape(128, 8, 8, 128).transpose(0, 2, 1, 3)
    table = _convert(tex5)
    o5 = _sample(table, u5, v5)
    return o5.transpose(0, 2, 4, 1, 3).reshape(_H, _W, _C)
